# Initial kernel scaffold; baseline (speedup 1.0000x reference)
import functools

import jax
import jax.numpy as jnp
from jax import lax
from jax.experimental import pallas as pl
from jax.experimental.pallas import tpu as pltpu
from jax.experimental.pallas import tpu_sc as plsc

LANES = 16
R = 64
R_LOG2 = 6


@functools.lru_cache(maxsize=None)
def _build(T, D, B, NC, NS):
    DCOL = D // NC
    KCH = DCOL // LANES

    mesh = plsc.VectorSubcoreMesh(core_axis_name="c", subcore_axis_name="s")

    @functools.partial(
        pl.kernel,
        mesh=mesh,
        out_type=jax.ShapeDtypeStruct((B, NC, DCOL), jnp.float32),
        scratch_types=[
            pltpu.VMEM((R, DCOL), jnp.float32),
            pltpu.VMEM((DCOL,), jnp.float32),
            pltpu.VMEM((NS,), jnp.int32),
            pltpu.VMEM((NS,), jnp.int32),
            pltpu.VMEM((NS,), jnp.float32),
        ],
    )
    def pool_kernel(hid, starts_h, lens_h, inv_h, out, buf, accv, st_v, ln_v, iv_v):
        c = lax.axis_index("c")
        s = lax.axis_index("s")
        pltpu.sync_copy(starts_h, st_v)
        pltpu.sync_copy(lens_h, ln_v)
        pltpu.sync_copy(inv_h, iv_v)

        lane = lax.iota(jnp.int32, LANES)
        sel = lane == s
        start = jnp.sum(jnp.where(sel, st_v[...], 0))
        seg_len = jnp.sum(jnp.where(sel, ln_v[...], 0))
        inv = jnp.sum(jnp.where(sel, iv_v[...], jnp.float32(0.0)))
        nch = (seg_len + (R - 1)) >> R_LOG2

        def chunk_body(i, acc):
            raw = start + i * R
            dstart = jnp.minimum(raw, T - R)
            shift = raw - dstart
            off = dstart - start
            pltpu.sync_copy(hid.at[pl.ds(dstart, R), c], buf)

            def row_body(j, acc):
                valid = (j >= shift) & (off + j < seg_len)
                w = jnp.where(valid, jnp.float32(1.0), jnp.float32(0.0))
                return tuple(
                    acc[k] + buf[j, pl.ds(k * LANES, LANES)] * w
                    for k in range(KCH)
                )

            return lax.fori_loop(0, R, row_body, acc)

        acc0 = tuple(jnp.zeros((LANES,), jnp.float32) for _ in range(KCH))
        acc = lax.fori_loop(0, nch, chunk_body, acc0)

        for k in range(KCH):
            accv[pl.ds(k * LANES, LANES)] = acc[k] * inv
        pltpu.sync_copy(accv, out.at[s, c])

    return pool_kernel


def kernel(hidden_states, cu_seqlens):
    T, D = hidden_states.shape
    B = cu_seqlens.shape[0] - 1
    info = plsc.get_sparse_core_info()
    NC, NS = info.num_cores, info.num_subcores

    starts = cu_seqlens[:-1]
    lens = cu_seqlens[1:] - cu_seqlens[:-1]
    inv = 1.0 / lens.astype(jnp.float32)

    hid3 = hidden_states.reshape(T, NC, D // NC)
    pooled = _build(T, D, B, NC, NS)(hid3, starts, lens, inv)
    return pooled.reshape(B, D)


# speedup vs baseline: 3.2471x; 3.2471x over previous
import functools

import jax
import jax.numpy as jnp
from jax import lax
from jax.experimental import pallas as pl
from jax.experimental.pallas import tpu as pltpu
from jax.experimental.pallas import tpu_sc as plsc

LANES = 16
R = 64
R_LOG2 = 6


@functools.lru_cache(maxsize=None)
def _build(T, D, B, NC, NS):
    DCOL = D // NC
    KCH = DCOL // LANES

    mesh = plsc.VectorSubcoreMesh(core_axis_name="c", subcore_axis_name="s")

    @functools.partial(
        pl.kernel,
        mesh=mesh,
        out_type=jax.ShapeDtypeStruct((B, NC, DCOL), jnp.float32),
        scratch_types=[
            pltpu.VMEM((R, DCOL), jnp.float32),
            pltpu.VMEM((DCOL,), jnp.float32),
            pltpu.VMEM((2 * LANES,), jnp.int32),
            pltpu.VMEM((2 * LANES,), jnp.int32),
            pltpu.VMEM((2 * LANES,), jnp.float32),
        ],
    )
    def pool_kernel(hid, starts_h, lens_h, inv_h, out, buf, accv, st_v, ln_v, iv_v):
        c = lax.axis_index("c")
        s = lax.axis_index("s")
        pltpu.sync_copy(starts_h, st_v.at[pl.ds(0, NS)])
        pltpu.sync_copy(lens_h, ln_v.at[pl.ds(0, NS)])
        pltpu.sync_copy(inv_h, iv_v.at[pl.ds(0, NS)])

        start = st_v[pl.ds(s, LANES)][0]
        seg_len = ln_v[pl.ds(s, LANES)][0]
        inv = iv_v[pl.ds(s, LANES)][0]
        nch = (seg_len + (R - 1)) >> R_LOG2

        def chunk_body(i, acc):
            raw = start + i * R
            dstart = jnp.minimum(raw, T - R)
            shift = raw - dstart
            off = dstart - start
            pltpu.sync_copy(hid.at[pl.ds(dstart, R), c], buf)

            def row_body(j, acc):
                valid = (j >= shift) & (off + j < seg_len)
                w = jnp.where(valid, jnp.float32(1.0), jnp.float32(0.0))
                return tuple(
                    acc[k] + buf[j, pl.ds(k * LANES, LANES)] * w
                    for k in range(KCH)
                )

            return lax.fori_loop(0, R, row_body, acc)

        acc0 = tuple(jnp.zeros((LANES,), jnp.float32) for _ in range(KCH))
        acc = lax.fori_loop(0, nch, chunk_body, acc0)

        for k in range(KCH):
            accv[pl.ds(k * LANES, LANES)] = acc[k] * inv
        pltpu.sync_copy(accv, out.at[s, c])

    return pool_kernel


def kernel(hidden_states, cu_seqlens):
    T, D = hidden_states.shape
    B = cu_seqlens.shape[0] - 1
    info = plsc.get_sparse_core_info()
    NC, NS = info.num_cores, info.num_subcores

    starts = cu_seqlens[:-1]
    lens = cu_seqlens[1:] - cu_seqlens[:-1]
    inv = 1.0 / lens.astype(jnp.float32)

    hid3 = hidden_states.reshape(T, NC, D // NC)
    pooled = _build(T, D, B, NC, NS)(hid3, starts, lens, inv)
    return pooled.reshape(B, D)


# speedup vs baseline: 3.4182x; 1.0527x over previous
import functools

import jax
import jax.numpy as jnp
from jax import lax
from jax.experimental import pallas as pl
from jax.experimental.pallas import tpu as pltpu
from jax.experimental.pallas import tpu_sc as plsc

LANES = 16
R = 64
R_LOG2 = 6
UROWS = 8


@functools.lru_cache(maxsize=None)
def _build(T, D, B, NC, NS):
    DCOL = D // NC
    KCH = DCOL // LANES

    mesh = plsc.VectorSubcoreMesh(core_axis_name="c", subcore_axis_name="s")

    @functools.partial(
        pl.kernel,
        mesh=mesh,
        out_type=jax.ShapeDtypeStruct((B, NC, DCOL), jnp.float32),
        scratch_types=[
            pltpu.VMEM((R, DCOL), jnp.float32),
            pltpu.VMEM((R, DCOL), jnp.float32),
            pltpu.VMEM((DCOL,), jnp.float32),
            pltpu.VMEM((2 * LANES,), jnp.int32),
            pltpu.VMEM((2 * LANES,), jnp.int32),
            pltpu.VMEM((2 * LANES,), jnp.float32),
            pltpu.SemaphoreType.DMA,
            pltpu.SemaphoreType.DMA,
        ],
    )
    def pool_kernel(hid, starts_h, lens_h, inv_h, out, buf0, buf1, accv,
                    st_v, ln_v, iv_v, sem0, sem1):
        c = lax.axis_index("c")
        s = lax.axis_index("s")
        pltpu.sync_copy(starts_h, st_v.at[pl.ds(0, NS)])
        pltpu.sync_copy(lens_h, ln_v.at[pl.ds(0, NS)])
        pltpu.sync_copy(inv_h, iv_v.at[pl.ds(0, NS)])

        start = st_v[pl.ds(s, LANES)][0]
        seg_len = ln_v[pl.ds(s, LANES)][0]
        inv = iv_v[pl.ds(s, LANES)][0]
        nch = (seg_len + (R - 1)) >> R_LOG2
        npairs = (nch + 1) >> 1
        nch_pad = npairs * 2

        def dma_start(i, buf, sem):
            raw = start + i * R
            dstart = jnp.minimum(raw, T - R)
            pltpu.async_copy(hid.at[pl.ds(dstart, R), c], buf, sem)

        def dma_wait(buf, sem):
            pltpu.make_async_copy(hid.at[pl.ds(0, R), c], buf, sem).wait()

        @pl.when(npairs > 0)
        def _():
            dma_start(0, buf0, sem0)
            dma_start(1, buf1, sem1)

        def accum_chunk(i, acc, buf, sem):
            dma_wait(buf, sem)
            raw = start + i * R
            dstart = jnp.minimum(raw, T - R)
            shift = raw - dstart
            off = dstart - start

            def u_body(u, acc):
                j0 = u * UROWS
                new = list(acc)
                for du in range(UROWS):
                    j = j0 + du
                    valid = (j >= shift) & (off + j < seg_len)
                    w = jnp.where(valid, jnp.float32(1.0), jnp.float32(0.0))
                    for k in range(KCH):
                        new[k] = new[k] + buf[j, pl.ds(k * LANES, LANES)] * w
                return tuple(new)

            acc = lax.fori_loop(0, R // UROWS, u_body, acc)

            @pl.when(i + 2 < nch_pad)
            def _():
                dma_start(i + 2, buf, sem)

            return acc

        def pair_body(p, acc):
            i = p * 2
            acc = accum_chunk(i, acc, buf0, sem0)
            acc = accum_chunk(i + 1, acc, buf1, sem1)
            return acc

        acc0 = tuple(jnp.zeros((LANES,), jnp.float32) for _ in range(KCH))
        acc = lax.fori_loop(0, npairs, pair_body, acc0)

        for k in range(KCH):
            accv[pl.ds(k * LANES, LANES)] = acc[k] * inv
        pltpu.sync_copy(accv, out.at[s, c])

    return pool_kernel


def kernel(hidden_states, cu_seqlens):
    T, D = hidden_states.shape
    B = cu_seqlens.shape[0] - 1
    info = plsc.get_sparse_core_info()
    NC, NS = info.num_cores, info.num_subcores

    starts = cu_seqlens[:-1]
    lens = cu_seqlens[1:] - cu_seqlens[:-1]
    inv = 1.0 / lens.astype(jnp.float32)

    hid3 = hidden_states.reshape(T, NC, D // NC)
    pooled = _build(T, D, B, NC, NS)(hid3, starts, lens, inv)
    return pooled.reshape(B, D)


# speedup vs baseline: 10.6604x; 3.1187x over previous
import functools

import jax
import jax.numpy as jnp
from jax import lax
from jax.experimental import pallas as pl
from jax.experimental.pallas import tpu as pltpu
from jax.experimental.pallas import tpu_sc as plsc

LANES = 16
R = 64
R_LOG2 = 6
UROWS = 8


@functools.lru_cache(maxsize=None)
def _build(T, D, B, NC, NS):
    DCOL = D // NC
    KCH = DCOL // LANES

    mesh = plsc.VectorSubcoreMesh(core_axis_name="c", subcore_axis_name="s")

    @functools.partial(
        pl.kernel,
        mesh=mesh,
        out_type=jax.ShapeDtypeStruct((B, NC, DCOL), jnp.float32),
        scratch_types=[
            pltpu.VMEM((R, DCOL), jnp.float32),
            pltpu.VMEM((R, DCOL), jnp.float32),
            pltpu.VMEM((DCOL,), jnp.float32),
            pltpu.VMEM((2 * LANES,), jnp.int32),
            pltpu.VMEM((2 * LANES,), jnp.int32),
            pltpu.VMEM((2 * LANES,), jnp.float32),
            pltpu.SemaphoreType.DMA,
            pltpu.SemaphoreType.DMA,
        ],
    )
    def pool_kernel(hid, starts_h, lens_h, inv_h, out, buf0, buf1, accv,
                    st_v, ln_v, iv_v, sem0, sem1):
        c = lax.axis_index("c")
        s = lax.axis_index("s")
        pltpu.sync_copy(starts_h, st_v.at[pl.ds(0, NS)])
        pltpu.sync_copy(lens_h, ln_v.at[pl.ds(0, NS)])
        pltpu.sync_copy(inv_h, iv_v.at[pl.ds(0, NS)])

        start = st_v[pl.ds(s, LANES)][0]
        seg_len = ln_v[pl.ds(s, LANES)][0]
        inv = iv_v[pl.ds(s, LANES)][0]
        abase = jnp.bitwise_and(start, ~7)
        sshift = start - abase
        nch = (seg_len + sshift + (R - 1)) >> R_LOG2
        npairs = (nch + 1) >> 1
        nch_pad = npairs * 2

        col0 = c * DCOL

        def dma_start(i, buf, sem):
            raw = abase + i * R
            dstart = pl.multiple_of(jnp.minimum(raw, T - R), 8)
            pltpu.async_copy(hid.at[pl.ds(dstart, R), pl.ds(col0, DCOL)], buf, sem)

        def dma_wait(buf, sem):
            pltpu.make_async_copy(hid.at[pl.ds(0, R), pl.ds(col0, DCOL)], buf, sem).wait()

        @pl.when(npairs > 0)
        def _():
            dma_start(0, buf0, sem0)
            dma_start(1, buf1, sem1)

        def accum_chunk(i, acc, buf, sem):
            dma_wait(buf, sem)
            raw = abase + i * R
            dstart = jnp.minimum(raw, T - R)
            shift = raw - dstart
            off = dstart - start

            def u_body(u, acc):
                j0 = u * UROWS
                new = list(acc)
                for du in range(UROWS):
                    j = j0 + du
                    jj = off + j
                    valid = (j >= shift) & (jj >= 0) & (jj < seg_len)
                    w = jnp.where(valid, jnp.float32(1.0), jnp.float32(0.0))
                    for k in range(KCH):
                        new[k] = new[k] + buf[j, pl.ds(k * LANES, LANES)] * w
                return tuple(new)

            acc = lax.fori_loop(0, R // UROWS, u_body, acc)

            @pl.when(i + 2 < nch_pad)
            def _():
                dma_start(i + 2, buf, sem)

            return acc

        def pair_body(p, acc):
            i = p * 2
            acc = accum_chunk(i, acc, buf0, sem0)
            acc = accum_chunk(i + 1, acc, buf1, sem1)
            return acc

        acc0 = tuple(jnp.zeros((LANES,), jnp.float32) for _ in range(KCH))
        acc = lax.fori_loop(0, npairs, pair_body, acc0)

        for k in range(KCH):
            accv[pl.ds(k * LANES, LANES)] = acc[k] * inv
        pltpu.sync_copy(accv, out.at[s, c])

    return pool_kernel


def kernel(hidden_states, cu_seqlens):
    T, D = hidden_states.shape
    B = cu_seqlens.shape[0] - 1
    info = plsc.get_sparse_core_info()
    NC, NS = info.num_cores, info.num_subcores

    starts = cu_seqlens[:-1]
    lens = cu_seqlens[1:] - cu_seqlens[:-1]
    inv = 1.0 / lens.astype(jnp.float32)

    pooled = _build(T, D, B, NC, NS)(hidden_states, starts, lens, inv)
    return pooled.reshape(B, D)
